# baseline (speedup 1.0000x reference)
import functools

import jax
import jax.numpy as jnp
import numpy as np
from jax.experimental import pallas as pl

_ACTIONS = 1000
_ROWS = 256

_TINY = float(np.finfo(np.float32).tiny)
_ONE_BITS = np.uint32(0x3F800000)
_KS = (np.uint32(0), np.uint32(42), np.uint32(0 ^ 42 ^ 0x1BD11BDA))
_ROT_A = (13, 15, 26, 6)
_ROT_B = (17, 29, 16, 24)


def _rotl(v, r):
    return (v << np.uint32(r)) | (v >> np.uint32(32 - r))


def _rounds(x0, x1, rots):
    for r in rots:
        x0 = x0 + x1
        x1 = _rotl(x1, r)
        x1 = x1 ^ x0
    return x0, x1


def _threefry_bits(j):
    x0 = jnp.full_like(j, _KS[0])
    x1 = j + _KS[1]
    x0, x1 = _rounds(x0, x1, _ROT_A)
    x0 = x0 + _KS[1]
    x1 = x1 + _KS[2] + np.uint32(1)
    x0, x1 = _rounds(x0, x1, _ROT_B)
    x0 = x0 + _KS[2]
    x1 = x1 + _KS[0] + np.uint32(2)
    x0, x1 = _rounds(x0, x1, _ROT_A)
    x0 = x0 + _KS[0]
    x1 = x1 + _KS[1] + np.uint32(3)
    x0, x1 = _rounds(x0, x1, _ROT_B)
    x0 = x0 + _KS[1]
    x1 = x1 + _KS[2] + np.uint32(4)
    x0, x1 = _rounds(x0, x1, _ROT_A)
    x0 = x0 + _KS[2]
    x1 = x1 + _KS[0] + np.uint32(5)
    return x0 ^ x1


def _block_kernel(logits_ref, sample_ref, logp_ref):
    i = pl.program_id(0)
    logits = logits_ref[...]
    rows, acts = logits.shape

    row = jax.lax.broadcasted_iota(jnp.uint32, (rows, acts), 0)
    col_i = jax.lax.broadcasted_iota(jnp.int32, (rows, acts), 1)
    base = (i * (rows * acts)).astype(jnp.uint32)
    j = base + row * np.uint32(acts) + col_i.astype(jnp.uint32)

    bits = _threefry_bits(j)
    f = jax.lax.bitcast_convert_type(
        (bits >> np.uint32(9)) | _ONE_BITS, jnp.float32) - 1.0
    u = jnp.where(f == 0.0, np.float32(_TINY), f)
    gumbel = -jnp.log(-jnp.log(u))

    s = logits + gumbel
    smax = jnp.max(s, axis=1, keepdims=True)
    cls = jnp.min(jnp.where(s == smax, col_i, acts), axis=1, keepdims=True)
    onehot = col_i == cls
    sample_ref[...] = onehot.astype(jnp.float32)

    lmax = jnp.max(logits, axis=1, keepdims=True)
    shifted = logits - lmax
    lse = jnp.log(jnp.sum(jnp.exp(shifted), axis=1, keepdims=True))
    picked = jnp.max(jnp.where(onehot, shifted, -jnp.inf), axis=1, keepdims=True)
    logp_ref[...] = picked - lse


@jax.jit
def kernel(logits):
    batch, acts = logits.shape
    grid = batch // _ROWS
    sample, logp = pl.pallas_call(
        _block_kernel,
        grid=(grid,),
        in_specs=[pl.BlockSpec((_ROWS, acts), lambda i: (i, 0))],
        out_specs=[
            pl.BlockSpec((_ROWS, acts), lambda i: (i, 0)),
            pl.BlockSpec((_ROWS, 1), lambda i: (i, 0)),
        ],
        out_shape=[
            jax.ShapeDtypeStruct((batch, acts), jnp.float32),
            jax.ShapeDtypeStruct((batch, 1), jnp.float32),
        ],
    )(logits)
    return (sample, logp)
